# baseline (speedup 1.0000x reference)
import jax
import jax.numpy as jnp
import numpy as np
from jax import lax
from jax.experimental import pallas as pl
from jax.experimental.pallas import tpu as pltpu
from jax.experimental.pallas import tpu_sc as plsc

_N_NODES = 10000
_D = 128
_N_POS = 320000
_N_NEG = 320000
_E = _N_POS + _N_NEG
_PROB = _N_POS / (_N_NODES**2 - _N_NODES) * 2
_EPS = float(-np.log(1.0 - _PROB))

_NC = 2
_NS = 16
_NW = _NC * _NS
_EPW = _E // _NW
_B = 32
_NBLK = _EPW // _B
_RING = 5
_LANES = 16


def _sc_scores_kernel(h_hbm, u_hbm, v_hbm, out_hbm, *scr):
    hsh = scr[0]
    iu = scr[1:1 + _RING]
    iv = scr[6:6 + _RING]
    ru = scr[11:11 + _RING]
    rv = scr[16:16 + _RING]
    sco = scr[21:21 + _RING]
    isem = scr[26:26 + _RING]
    gsem = scr[31:31 + _RING]
    ssem = scr[36:36 + _RING]

    sid = lax.axis_index("s")
    wid = sid * _NC + lax.axis_index("c")
    base = wid * _EPW

    rows_per_sub = 624
    pltpu.sync_copy(h_hbm.at[pl.ds(sid * rows_per_sub, rows_per_sub)],
                    hsh.at[pl.ds(sid * rows_per_sub, rows_per_sub)])

    @pl.when(sid == _NS - 1)
    def _():
        pltpu.sync_copy(h_hbm.at[pl.ds(_NS * rows_per_sub, 16)],
                        hsh.at[pl.ds(_NS * rows_per_sub, 16)])

    plsc.subcore_barrier()

    def issue_idx(blk, slot):
        off = base + blk * _B
        pltpu.async_copy(u_hbm.at[pl.ds(off, _B)], iu[slot], isem[slot])
        pltpu.async_copy(v_hbm.at[pl.ds(off, _B)], iv[slot], isem[slot])

    def wait_idx(slot):
        pltpu.make_async_copy(u_hbm.at[pl.ds(0, _B)], iu[slot], isem[slot]).wait()
        pltpu.make_async_copy(v_hbm.at[pl.ds(0, _B)], iv[slot], isem[slot]).wait()

    def issue_gather(slot):
        pltpu.async_copy(hsh.at[iu[slot]], ru[slot], gsem[slot])
        pltpu.async_copy(hsh.at[iv[slot]], rv[slot], gsem[slot])

    def wait_gather(slot):
        pltpu.make_async_copy(hsh.at[iu[slot]], ru[slot], gsem[slot]).wait()
        pltpu.make_async_copy(hsh.at[iv[slot]], rv[slot], gsem[slot]).wait()

    def wait_score(slot):
        pltpu.make_async_copy(sco[slot], out_hbm.at[pl.ds(0, _B)], ssem[slot]).wait()

    for k in range(_RING):
        issue_idx(k, k)
    for k in range(_RING - 1):
        wait_idx(k)
        issue_gather(k)

    def compute(blk, slot):
        def g_body(g, carry):
            ev = lax.iota(jnp.int32, _LANES) + g * _LANES
            acc = jnp.zeros((_LANES,), jnp.float32)
            for d in range(_D):
                dv = jnp.full((_LANES,), d, jnp.int32)
                uu = plsc.load_gather(ru[slot], [ev, dv])
                vv = plsc.load_gather(rv[slot], [ev, dv])
                acc = acc + uu * vv
            sco[slot][pl.ds(g * _LANES, _LANES)] = acc
            return carry
        lax.fori_loop(0, _B // _LANES, g_body, 0)

    def j_body(j, carry):
        for b in range(_RING):
            i = j * _RING + b
            wait_gather(b)

            @pl.when(i + _RING < _NBLK)
            def _():
                issue_idx(i + _RING, b)

            @pl.when(i + _RING - 1 < _NBLK)
            def _():
                wait_idx((b + _RING - 1) % _RING)
                issue_gather((b + _RING - 1) % _RING)

            @pl.when(i >= _RING)
            def _():
                wait_score(b)

            compute(i, b)
            pltpu.async_copy(sco[b], out_hbm.at[pl.ds(base + i * _B, _B)],
                             ssem[b])
        return carry

    lax.fori_loop(0, _NBLK // _RING, j_body, 0)

    for k in range(_RING):
        wait_score(k)


@jax.jit
def _sc_scores(h, u, v):
    mesh = plsc.VectorSubcoreMesh(core_axis_name="c", subcore_axis_name="s")
    scratch = [pltpu.VMEM_SHARED((_N_NODES, _D), jnp.float32)]
    scratch += [pltpu.VMEM((_B,), jnp.int32) for _ in range(2 * _RING)]
    scratch += [pltpu.VMEM((_B, _D), jnp.float32) for _ in range(2 * _RING)]
    scratch += [pltpu.VMEM((_B,), jnp.float32) for _ in range(_RING)]
    scratch += [pltpu.SemaphoreType.DMA for _ in range(3 * _RING)]
    return pl.kernel(
        _sc_scores_kernel,
        out_type=jax.ShapeDtypeStruct((_E,), jnp.float32),
        mesh=mesh,
        compiler_params=pltpu.CompilerParams(needs_layout_passes=False),
        scratch_types=scratch,
    )(h, u, v)


def _loss_body(s_ref, o_ref):
    s = s_ref[...]
    pos = s[: _N_POS // _D, :]
    neg = s[_N_POS // _D:, :]
    t = jnp.exp(-pos - _EPS)
    loss_edges = jnp.mean(jnp.log(1.0 + t))
    loss_nonedges = jnp.mean(neg)
    o_ref[...] = jnp.reshape(loss_edges + loss_nonedges, (1, 1))


@jax.jit
def _tc_loss(scores2d):
    out = pl.pallas_call(
        _loss_body,
        out_shape=jax.ShapeDtypeStruct((1, 1), jnp.float32),
    )(scores2d)
    return out[0, 0]


def kernel(block_outputs, pos_edge_index, neg_edge_index):
    u = jnp.concatenate([pos_edge_index[0], neg_edge_index[0]])
    v = jnp.concatenate([pos_edge_index[1], neg_edge_index[1]])
    scores = _sc_scores(block_outputs, u, v)
    return _tc_loss(scores.reshape(_E // _D, _D))


# speedup vs baseline: 2.4376x; 2.4376x over previous
import jax
import jax.numpy as jnp
import numpy as np
from jax import lax
from jax.experimental import pallas as pl
from jax.experimental.pallas import tpu as pltpu
from jax.experimental.pallas import tpu_sc as plsc

_N_NODES = 10000
_D = 128
_N_POS = 320000
_N_NEG = 320000
_E = _N_POS + _N_NEG
_PROB = _N_POS / (_N_NODES**2 - _N_NODES) * 2
_EPS = float(-np.log(1.0 - _PROB))

_NC = 2
_NS = 16
_FPT = _D // _NS
_CH = 4096
_NCH = 80
_EPC = _CH * _NCH
_CROWS = _CH // _D
_SROWS = _NCH * _CROWS
_WB = _SROWS // _NS
_LANES = 16
_VBITS = 14


def _sc_scores_kernel(ht_hbm, pk_hbm, out_hbm,
                      tsl, pk0, pk1, s0, s1, io, shsc,
                      isem0, isem1, ssem0, ssem1):
    sid = lax.axis_index("s")
    cid = lax.axis_index("c")

    pltpu.sync_copy(ht_hbm.at[pl.ds(sid * _N_NODES * _FPT, _N_NODES * _FPT)],
                    tsl)

    def io_body(j, carry):
        for k in range(_CROWS // _LANES):
            io[j, pl.ds(k * _LANES, _LANES)] = (
                lax.iota(jnp.int32, _LANES) + j * _CROWS + k * _LANES)
        return carry
    lax.fori_loop(0, _NCH, io_body, 0)

    def store_block(scb, g, val):
        scb[g // (_D // _LANES),
            pl.ds((g % (_D // _LANES)) * _LANES, _LANES)] = val

    def z_body(g, carry):
        store_block(s0, g, jnp.zeros((_LANES,), jnp.float32))
        return carry
    lax.fori_loop(0, _CH // _LANES, z_body, 0)
    for k in range(_WB // _CROWS):
        pltpu.sync_copy(s0, shsc.at[pl.ds(sid * _WB + k * _CROWS, _CROWS)])
    plsc.subcore_barrier()

    def issue_idx(chunk, pkb, isem):
        off = cid * _EPC + chunk * _CH
        pltpu.async_copy(pk_hbm.at[pl.ds(off, _CH)], pkb, isem)

    def wait_idx(pkb, isem):
        pltpu.make_async_copy(pk_hbm.at[pl.ds(0, _CH)], pkb, isem).wait()

    def wait_sadd(scb, ssem):
        pltpu.make_async_copy(scb, shsc.at[io.at[0]], ssem).wait()

    issue_idx(0, pk0, isem0)
    issue_idx(1, pk1, isem1)

    def compute(pkb, scb):
        def g_body(g, carry):
            pk16 = pkb[pl.ds(g * _LANES, _LANES)]
            u8 = lax.shift_left(lax.shift_right_logical(pk16, _VBITS), 3)
            v8 = lax.shift_left(
                lax.bitwise_and(pk16, (1 << _VBITS) - 1), 3)
            acc = jnp.zeros((_LANES,), jnp.float32)
            for f in range(_FPT):
                acc = acc + (plsc.load_gather(tsl, [u8 + f])
                             * plsc.load_gather(tsl, [v8 + f]))
            store_block(scb, g, acc)
            return carry
        lax.fori_loop(0, _CH // _LANES, g_body, 0)

    bufs = ((pk0, s0, isem0, ssem0), (pk1, s1, isem1, ssem1))

    def j_body(jj, carry):
        for b in range(2):
            j = jj * 2 + b
            pkb, scb, isem, ssem = bufs[b]
            wait_idx(pkb, isem)

            @pl.when(j >= 2)
            def _():
                wait_sadd(scb, ssem)

            compute(pkb, scb)
            pltpu.async_copy(scb, shsc.at[io.at[j]], ssem, add=True)

            @pl.when(j + 2 < _NCH)
            def _():
                issue_idx(j + 2, pkb, isem)
        return carry

    lax.fori_loop(0, _NCH // 2, j_body, 0)

    wait_sadd(s0, ssem0)
    wait_sadd(s1, ssem1)
    plsc.subcore_barrier()

    pltpu.sync_copy(shsc.at[pl.ds(sid * _WB, _WB)],
                    out_hbm.at[cid, pl.ds(sid * _WB, _WB)])


@jax.jit
def _sc_scores(ht, pk):
    mesh = plsc.VectorSubcoreMesh(core_axis_name="c", subcore_axis_name="s")
    return pl.kernel(
        _sc_scores_kernel,
        out_type=jax.ShapeDtypeStruct((_NC, _SROWS, _D), jnp.float32),
        mesh=mesh,
        compiler_params=pltpu.CompilerParams(needs_layout_passes=False),
        scratch_types=[
            pltpu.VMEM((_N_NODES * _FPT,), jnp.float32),
            pltpu.VMEM((_CH,), jnp.int32),
            pltpu.VMEM((_CH,), jnp.int32),
            pltpu.VMEM((_CROWS, _D), jnp.float32),
            pltpu.VMEM((_CROWS, _D), jnp.float32),
            pltpu.VMEM((_NCH, _CROWS), jnp.int32),
            pltpu.VMEM_SHARED((_SROWS, _D), jnp.float32),
            pltpu.SemaphoreType.DMA,
            pltpu.SemaphoreType.DMA,
            pltpu.SemaphoreType.DMA,
            pltpu.SemaphoreType.DMA,
        ],
    )(ht, pk)


def _loss_body(s_ref, o_ref):
    s = s_ref[...]
    pos = s[: _N_POS // _D, :]
    neg = s[_N_POS // _D:, :]
    t = jnp.exp(-pos - _EPS)
    loss_edges = jnp.mean(jnp.log(1.0 + t))
    loss_nonedges = jnp.mean(neg)
    o_ref[...] = jnp.reshape(loss_edges + loss_nonedges, (1, 1))


@jax.jit
def _tc_loss(scores2d):
    out = pl.pallas_call(
        _loss_body,
        out_shape=jax.ShapeDtypeStruct((1, 1), jnp.float32),
    )(scores2d)
    return out[0, 0]


def kernel(block_outputs, pos_edge_index, neg_edge_index):
    u = jnp.concatenate([pos_edge_index[0], neg_edge_index[0]])
    v = jnp.concatenate([pos_edge_index[1], neg_edge_index[1]])
    pk = u * (1 << _VBITS) + v
    pad = jnp.zeros((_EPC - _E // _NC,), jnp.int32)
    pkp = jnp.concatenate([pk[: _E // _NC], pad, pk[_E // _NC:], pad])
    ht = block_outputs.reshape(_N_NODES, _NS, _FPT).transpose(1, 0, 2)
    ht = ht.reshape(_NS * _N_NODES * _FPT)
    scores = _sc_scores(ht, pkp)
    real = scores[:, : (_E // _NC) // _D, :].reshape(_E // _D, _D)
    return _tc_loss(real)


# speedup vs baseline: 4.4521x; 1.8265x over previous
import jax
import jax.numpy as jnp
import numpy as np
from jax import lax
from jax.experimental import pallas as pl
from jax.experimental.pallas import tpu as pltpu
from jax.experimental.pallas import tpu_sc as plsc

_N_NODES = 10000
_D = 128
_N_POS = 320000
_N_NEG = 320000
_E = _N_POS + _N_NEG
_PROB = _N_POS / (_N_NODES**2 - _N_NODES) * 2
_EPS = float(-np.log(1.0 - _PROB))

_NC = 2
_NS = 16
_FPT = _D // _NS
_CH = 4096
_NCH = 80
_EPC = _CH * _NCH
_CROWS = _CH // _D
_SROWS = _NCH * _CROWS
_WB = _SROWS // _NS
_LANES = 16
_VBITS = 14


def _sc_scores_kernel(ht_hbm, pk_hbm, out_hbm,
                      tsl, pk0, pk1, s0, s1, io, shsc,
                      isem0, isem1, ssem0, ssem1):
    sid = lax.axis_index("s")
    cid = lax.axis_index("c")

    pltpu.sync_copy(ht_hbm.at[pl.ds(sid * _N_NODES * _FPT, _N_NODES * _FPT)],
                    tsl)

    def io_body(j, carry):
        for k in range(_CROWS // _LANES):
            io[j, pl.ds(k * _LANES, _LANES)] = (
                lax.iota(jnp.int32, _LANES) + j * _CROWS + k * _LANES)
        return carry
    lax.fori_loop(0, _NCH, io_body, 0)

    def store_block(scb, g, val):
        scb[g // (_D // _LANES),
            pl.ds((g % (_D // _LANES)) * _LANES, _LANES)] = val

    def z_body(g, carry):
        store_block(s0, g, jnp.zeros((_LANES,), jnp.float32))
        return carry
    lax.fori_loop(0, _CH // _LANES, z_body, 0)
    for k in range(_WB // _CROWS):
        pltpu.sync_copy(s0, shsc.at[pl.ds(sid * _WB + k * _CROWS, _CROWS)])
    plsc.subcore_barrier()

    def issue_idx(chunk, pkb, isem):
        off = cid * _EPC + chunk * _CH
        pltpu.async_copy(pk_hbm.at[pl.ds(off, _CH)], pkb, isem)

    def wait_idx(pkb, isem):
        pltpu.make_async_copy(pk_hbm.at[pl.ds(0, _CH)], pkb, isem).wait()

    def wait_sadd(scb, ssem):
        pltpu.make_async_copy(scb, shsc.at[io.at[0]], ssem).wait()

    issue_idx(0, pk0, isem0)
    issue_idx(1, pk1, isem1)

    def compute(pkb, scb):
        def g_body(g, carry):
            pk16 = pkb[pl.ds(g * _LANES, _LANES)]
            uvec = lax.shift_right_logical(pk16, _VBITS)
            vvec = lax.bitwise_and(pk16, (1 << _VBITS) - 1)
            acc = jnp.zeros((_LANES,), jnp.float32)
            for f in range(_FPT):
                acc = acc + (plsc.load_gather(tsl, [uvec + f * _N_NODES])
                             * plsc.load_gather(tsl, [vvec + f * _N_NODES]))
            store_block(scb, g, acc)
            return carry
        lax.fori_loop(0, _CH // _LANES, g_body, 0)

    bufs = ((pk0, s0, isem0, ssem0), (pk1, s1, isem1, ssem1))

    def j_body(jj, carry):
        for b in range(2):
            j = jj * 2 + b
            pkb, scb, isem, ssem = bufs[b]
            wait_idx(pkb, isem)

            @pl.when(j >= 2)
            def _():
                wait_sadd(scb, ssem)

            compute(pkb, scb)
            pltpu.async_copy(scb, shsc.at[io.at[j]], ssem, add=True)

            @pl.when(j + 2 < _NCH)
            def _():
                issue_idx(j + 2, pkb, isem)
        return carry

    lax.fori_loop(0, _NCH // 2, j_body, 0)

    wait_sadd(s0, ssem0)
    wait_sadd(s1, ssem1)
    plsc.subcore_barrier()

    pltpu.sync_copy(shsc.at[pl.ds(sid * _WB, _WB)],
                    out_hbm.at[cid, pl.ds(sid * _WB, _WB)])


@jax.jit
def _sc_scores(ht, pk):
    mesh = plsc.VectorSubcoreMesh(core_axis_name="c", subcore_axis_name="s")
    return pl.kernel(
        _sc_scores_kernel,
        out_type=jax.ShapeDtypeStruct((_NC, _SROWS, _D), jnp.float32),
        mesh=mesh,
        compiler_params=pltpu.CompilerParams(needs_layout_passes=False),
        scratch_types=[
            pltpu.VMEM((_N_NODES * _FPT,), jnp.float32),
            pltpu.VMEM((_CH,), jnp.int32),
            pltpu.VMEM((_CH,), jnp.int32),
            pltpu.VMEM((_CROWS, _D), jnp.float32),
            pltpu.VMEM((_CROWS, _D), jnp.float32),
            pltpu.VMEM((_NCH, _CROWS), jnp.int32),
            pltpu.VMEM_SHARED((_SROWS, _D), jnp.float32),
            pltpu.SemaphoreType.DMA,
            pltpu.SemaphoreType.DMA,
            pltpu.SemaphoreType.DMA,
            pltpu.SemaphoreType.DMA,
        ],
    )(ht, pk)


def _loss_body(s_ref, o_ref):
    s = s_ref[...]
    pos = s[: _N_POS // _D, :]
    neg = s[_N_POS // _D:, :]
    t = jnp.exp(-pos - _EPS)
    loss_edges = jnp.mean(jnp.log(1.0 + t))
    loss_nonedges = jnp.mean(neg)
    o_ref[...] = jnp.reshape(loss_edges + loss_nonedges, (1, 1))


@jax.jit
def _tc_loss(scores2d):
    out = pl.pallas_call(
        _loss_body,
        out_shape=jax.ShapeDtypeStruct((1, 1), jnp.float32),
    )(scores2d)
    return out[0, 0]


def kernel(block_outputs, pos_edge_index, neg_edge_index):
    u = jnp.concatenate([pos_edge_index[0], neg_edge_index[0]])
    v = jnp.concatenate([pos_edge_index[1], neg_edge_index[1]])
    pk = u * (1 << _VBITS) + v
    pad = jnp.zeros((_EPC - _E // _NC,), jnp.int32)
    pkp = jnp.concatenate([pk[: _E // _NC], pad, pk[_E // _NC:], pad])
    ht = block_outputs.T.reshape(_NS * _N_NODES * _FPT)
    scores = _sc_scores(ht, pkp)
    real = scores[:, : (_E // _NC) // _D, :].reshape(_E // _D, _D)
    return _tc_loss(real)


# speedup vs baseline: 7.7954x; 1.7509x over previous
import jax
import jax.numpy as jnp
import numpy as np
from jax import lax
from jax.experimental import pallas as pl
from jax.experimental.pallas import tpu as pltpu
from jax.experimental.pallas import tpu_sc as plsc

_N_NODES = 10000
_D = 128
_N_POS = 320000
_N_NEG = 320000
_E = _N_POS + _N_NEG
_PROB = _N_POS / (_N_NODES**2 - _N_NODES) * 2
_EPS = float(-np.log(1.0 - _PROB))

_NC = 2
_NS = 16
_FPT = _D // _NS
_CH = 4096
_NCH = 80
_EPC = _CH * _NCH
_CROWS = _CH // _D
_SROWS = _NCH * _CROWS
_WB = _SROWS // _NS
_LANES = 16
_VBITS = 14


def _sc_scores_kernel(ht_hbm, pk_hbm, out_hbm,
                      tsl, pk0, pk1, s0, s1, io, shsc,
                      isem0, isem1, ssem0, ssem1):
    sid = lax.axis_index("s")
    cid = lax.axis_index("c")

    pltpu.sync_copy(ht_hbm.at[pl.ds(sid * _N_NODES * _FPT, _N_NODES * _FPT)],
                    tsl)

    def io_body(j, carry):
        for k in range(_CROWS // _LANES):
            io[j, pl.ds(k * _LANES, _LANES)] = (
                lax.iota(jnp.int32, _LANES) + j * _CROWS + k * _LANES)
        return carry
    lax.fori_loop(0, _NCH, io_body, 0)

    def store_block(scb, g, val):
        scb[g // (_D // _LANES),
            pl.ds((g % (_D // _LANES)) * _LANES, _LANES)] = val

    def z_body(g, carry):
        store_block(s0, g, jnp.zeros((_LANES,), jnp.float32))
        return carry
    lax.fori_loop(0, _CH // _LANES, z_body, 0)
    for k in range(_WB // _CROWS):
        pltpu.sync_copy(s0, shsc.at[pl.ds(sid * _WB + k * _CROWS, _CROWS)])
    plsc.subcore_barrier()

    def issue_idx(chunk, pkb, isem):
        off = cid * _EPC + chunk * _CH
        pltpu.async_copy(pk_hbm.at[pl.ds(off, _CH)], pkb, isem)

    def wait_idx(pkb, isem):
        pltpu.make_async_copy(pk_hbm.at[pl.ds(0, _CH)], pkb, isem).wait()

    def wait_sadd(scb, ssem):
        pltpu.make_async_copy(scb, shsc.at[io.at[0]], ssem).wait()

    issue_idx(0, pk0, isem0)
    issue_idx(1, pk1, isem1)

    def compute(pkb, scb):
        @plsc.parallel_loop(0, _CH // _LANES, 1, unroll=2)
        def g_body(g):
            pk16 = pkb[pl.ds(g * _LANES, _LANES)]
            uvec = lax.shift_right_logical(pk16, _VBITS)
            vvec = lax.bitwise_and(pk16, (1 << _VBITS) - 1)
            acc = jnp.zeros((_LANES,), jnp.float32)
            for f in range(_FPT):
                acc = acc + (plsc.load_gather(tsl, [uvec + f * _N_NODES])
                             * plsc.load_gather(tsl, [vvec + f * _N_NODES]))
            store_block(scb, g, acc)

    bufs = ((pk0, s0, isem0, ssem0), (pk1, s1, isem1, ssem1))

    def j_body(jj, carry):
        for b in range(2):
            j = jj * 2 + b
            pkb, scb, isem, ssem = bufs[b]
            wait_idx(pkb, isem)

            @pl.when(j >= 2)
            def _():
                wait_sadd(scb, ssem)

            compute(pkb, scb)
            pltpu.async_copy(scb, shsc.at[io.at[j]], ssem, add=True)

            @pl.when(j + 2 < _NCH)
            def _():
                issue_idx(j + 2, pkb, isem)
        return carry

    lax.fori_loop(0, _NCH // 2, j_body, 0)

    wait_sadd(s0, ssem0)
    wait_sadd(s1, ssem1)
    plsc.subcore_barrier()

    pltpu.sync_copy(shsc.at[pl.ds(sid * _WB, _WB)],
                    out_hbm.at[cid, pl.ds(sid * _WB, _WB)])


@jax.jit
def _sc_scores(ht, pk):
    mesh = plsc.VectorSubcoreMesh(core_axis_name="c", subcore_axis_name="s")
    return pl.kernel(
        _sc_scores_kernel,
        out_type=jax.ShapeDtypeStruct((_NC, _SROWS, _D), jnp.float32),
        mesh=mesh,
        compiler_params=pltpu.CompilerParams(needs_layout_passes=False),
        scratch_types=[
            pltpu.VMEM((_N_NODES * _FPT,), jnp.float32),
            pltpu.VMEM((_CH,), jnp.int32),
            pltpu.VMEM((_CH,), jnp.int32),
            pltpu.VMEM((_CROWS, _D), jnp.float32),
            pltpu.VMEM((_CROWS, _D), jnp.float32),
            pltpu.VMEM((_NCH, _CROWS), jnp.int32),
            pltpu.VMEM_SHARED((_SROWS, _D), jnp.float32),
            pltpu.SemaphoreType.DMA,
            pltpu.SemaphoreType.DMA,
            pltpu.SemaphoreType.DMA,
            pltpu.SemaphoreType.DMA,
        ],
    )(ht, pk)


def _loss_body(s_ref, o_ref):
    s = s_ref[...]
    pos = s[: _N_POS // _D, :]
    neg = s[_N_POS // _D:, :]
    t = jnp.exp(-pos - _EPS)
    loss_edges = jnp.mean(jnp.log(1.0 + t))
    loss_nonedges = jnp.mean(neg)
    o_ref[...] = jnp.reshape(loss_edges + loss_nonedges, (1, 1))


@jax.jit
def _tc_loss(scores2d):
    out = pl.pallas_call(
        _loss_body,
        out_shape=jax.ShapeDtypeStruct((1, 1), jnp.float32),
    )(scores2d)
    return out[0, 0]


def kernel(block_outputs, pos_edge_index, neg_edge_index):
    u = jnp.concatenate([pos_edge_index[0], neg_edge_index[0]])
    v = jnp.concatenate([pos_edge_index[1], neg_edge_index[1]])
    pk = u * (1 << _VBITS) + v
    pad = jnp.zeros((_EPC - _E // _NC,), jnp.int32)
    pkp = jnp.concatenate([pk[: _E // _NC], pad, pk[_E // _NC:], pad])
    ht = block_outputs.T.reshape(_NS * _N_NODES * _FPT)
    scores = _sc_scores(ht, pkp)
    real = scores[:, : (_E // _NC) // _D, :].reshape(_E // _D, _D)
    return _tc_loss(real)
